# Initial kernel scaffold; baseline (speedup 1.0000x reference)
import jax
import jax.numpy as jnp
from jax.experimental import pallas as pl


def kernel(inputs, params):
    raise NotImplementedError("write your pallas kernel here")


# speedup vs baseline: 31.9329x; 31.9329x over previous
import jax
import jax.numpy as jnp
from jax import lax
from jax.experimental import pallas as pl
from jax.experimental.pallas import tpu as pltpu
from jax.experimental.pallas import tpu_sc as plsc

B, S, D = 4096, 50, 64
NC, NS, L = 2, 16, 16
NW = NC * NS
R = B // NW
G = R // L
C = D // L


def _body(in_hbm, par_hbm, out_hbm, in_v, p0_v, out_v):
    wid = lax.axis_index("s") * NC + lax.axis_index("c")
    base = wid * R
    pltpu.sync_copy(in_hbm.at[pl.ds(base, R)], in_v)
    pltpu.sync_copy(par_hbm.at[0], p0_v)

    iota = lax.iota(jnp.int32, L)
    accs = []
    for g in range(G):
        rows = iota + (g * L)
        acc = jnp.zeros((L,), jnp.float32)
        for l in range(S):
            cols = jnp.full((L,), l, jnp.int32)
            acc = acc + plsc.load_gather(in_v, [rows, cols])
        accs.append(acc)

    pcs = [p0_v[pl.ds(c * L, L)] for c in range(C)]
    for r in range(R):
        s = accs[r // L][r % L]
        for c in range(C):
            out_v[r, pl.ds(c * L, L)] = s * pcs[c]

    pltpu.sync_copy(out_v, out_hbm.at[pl.ds(base, R)])


@jax.jit
def kernel(inputs, params):
    mesh = plsc.VectorSubcoreMesh(
        core_axis_name="c", subcore_axis_name="s",
        num_cores=NC, num_subcores=NS,
    )
    out = pl.kernel(
        _body,
        out_type=jax.ShapeDtypeStruct((B, D), jnp.float32),
        mesh=mesh,
        compiler_params=pltpu.CompilerParams(needs_layout_passes=False),
        scratch_types=[
            pltpu.VMEM((R, S), jnp.float32),
            pltpu.VMEM((D,), jnp.float32),
            pltpu.VMEM((R, D), jnp.float32),
        ],
    )(inputs, params)
    return out[:, None, :]
